# baseline (speedup 1.0000x reference)
import functools

import jax
import jax.numpy as jnp
from jax.experimental import pallas as pl
from jax.experimental.pallas import tpu as pltpu

TB = 512
VMEM_LIMIT_BYTES = 96 * 1024 * 1024


def _fwd_kernel(x_ref, t1_ref, t2_ref, b2_ref,
                w1_ref, fb1_ref, w2_ref, fb2_ref,
                o_ref, a1_scr):
    f32 = jnp.float32
    bf16 = jnp.bfloat16

    lhs = jnp.concatenate(
        [x_ref[kh:kh + 32].reshape(32 * TB, 128) for kh in range(3)], axis=1)
    y = jnp.dot(lhs, t1_ref[...], preferred_element_type=f32)
    yh = jnp.maximum(y[:, :256], y[:, 256:]).reshape(16, 2 * TB, 256)
    a1 = jnp.maximum(jnp.maximum(yh[:, :TB], yh[:, TB:]), 0.0)
    a1_scr[1:17] = a1.astype(bf16)
    zpad = jnp.zeros((1, TB, 256), bf16)
    a1_scr[0:1] = zpad
    a1_scr[17:18] = zpad

    acc2 = jnp.dot(a1_scr[0:16].reshape(16 * TB, 256), t2_ref[0],
                   preferred_element_type=f32)
    for kh in (1, 2):
        acc2 = acc2 + jnp.dot(a1_scr[kh:kh + 16].reshape(16 * TB, 256),
                              t2_ref[kh], preferred_element_type=f32)
    y2 = acc2 + b2_ref[...]
    yh2 = jnp.maximum(y2[:, :128], y2[:, 128:]).reshape(8, 2 * TB, 128)
    pooled = jnp.maximum(jnp.maximum(yh2[:, :TB], yh2[:, TB:]),
                         0.0).astype(bf16)

    feats = jnp.concatenate([pooled[i] for i in range(8)], axis=1)
    h1 = jnp.maximum(
        jnp.dot(feats, w1_ref[...], preferred_element_type=f32) + fb1_ref[...],
        0.0)
    out = jnp.dot(h1.astype(bf16), w2_ref[...],
                  preferred_element_type=f32) + fb2_ref[...]
    o_ref[...] = out.astype(o_ref.dtype)


@jax.jit
def _forward(x_nchw, t1, b1, t2, b2, w1, fb1, w2, fb2):
    B, Cin, H, W = x_nchw.shape
    Bp = -(-B // TB) * TB

    t1p = jnp.pad(t1, ((0, 0), (0, 32), (0, 0)))
    t1p = t1p.at[0, 96, :].set(b1[0]).reshape(384, 512)
    ws = (t1p.astype(jnp.bfloat16), t2.astype(jnp.bfloat16), b2,
          w1.astype(jnp.bfloat16), fb1, w2.astype(jnp.bfloat16), fb2)

    xp = jnp.pad(x_nchw, ((0, Bp - B), (0, 0), (0, 0), (0, 0)))

    n_chunks = max(1, Bp // TB // 2)
    C = Bp // n_chunks

    def transform(xc):
        xh = jnp.transpose(xc, (2, 0, 3, 1)).reshape(H, C, W * Cin)
        xh = jnp.pad(xh, ((1, 1), (0, 0), (0, 0)))
        return jnp.concatenate(
            [xh, jnp.ones((H + 2, C, 1), xh.dtype),
             jnp.zeros((H + 2, C, 31), xh.dtype)], axis=2).astype(jnp.bfloat16)

    call = pl.pallas_call(
        _fwd_kernel,
        out_shape=jax.ShapeDtypeStruct((C, 128), jnp.float32),
        grid=(C // TB,),
        in_specs=[
            pl.BlockSpec((H + 2, TB, 128), lambda i: (0, i, 0)),
            pl.BlockSpec((384, 512), lambda i: (0, 0)),
            pl.BlockSpec((3, 256, 256), lambda i: (0, 0, 0)),
            pl.BlockSpec((1, 256), lambda i: (0, 0)),
            pl.BlockSpec((1024, 256), lambda i: (0, 0)),
            pl.BlockSpec((1, 256), lambda i: (0, 0)),
            pl.BlockSpec((256, 128), lambda i: (0, 0)),
            pl.BlockSpec((1, 128), lambda i: (0, 0)),
        ],
        out_specs=pl.BlockSpec((TB, 128), lambda i: (i, 0)),
        scratch_shapes=[pltpu.VMEM((18, TB, 256), jnp.bfloat16)],
        compiler_params=pltpu.CompilerParams(
            dimension_semantics=("parallel",),
            vmem_limit_bytes=VMEM_LIMIT_BYTES),
    )
    outs = [call(transform(xp[c * C:(c + 1) * C]), *ws)
            for c in range(n_chunks)]
    out = outs[0] if n_chunks == 1 else jnp.concatenate(outs, axis=0)
    return out[:B, :10]


def kernel(x, t1, b1, t2, b2, w1, fb1, w2, fb2):
    return _forward(x, t1, b1, t2, b2, w1, fb1, w2, fb2)


# speedup vs baseline: 1.4214x; 1.4214x over previous
import functools

import jax
import jax.numpy as jnp
from jax.experimental import pallas as pl
from jax.experimental.pallas import tpu as pltpu

TB = 512
VMEM_LIMIT_BYTES = 96 * 1024 * 1024


def _fwd_kernel(x_ref, t1_ref, t2_ref, b2_ref,
                w1_ref, fb1_ref, w2_ref, fb2_ref,
                o_ref, a1_scr):
    f32 = jnp.float32
    bf16 = jnp.bfloat16

    lhs = jnp.concatenate(
        [x_ref[kh:kh + 32].reshape(32 * TB, 128) for kh in range(3)], axis=1)
    zpad = jnp.zeros((1, TB, 256), bf16)
    a1_scr[0:1] = zpad
    a1_scr[17:18] = zpad
    for half in range(2):
        y = jnp.dot(lhs[half * 16 * TB:(half + 1) * 16 * TB], t1_ref[...],
                    preferred_element_type=f32)
        yh = jnp.maximum(y[:, :256], y[:, 256:]).reshape(8, 2 * TB, 256)
        a1 = jnp.maximum(jnp.maximum(yh[:, :TB], yh[:, TB:]), 0.0)
        a1_scr[1 + 8 * half:9 + 8 * half] = a1.astype(bf16)

    pooled_halves = []
    for half in range(2):
        h0 = 8 * half
        acc2 = jnp.dot(a1_scr[h0:h0 + 8].reshape(8 * TB, 256), t2_ref[0],
                       preferred_element_type=f32)
        for kh in (1, 2):
            acc2 = acc2 + jnp.dot(
                a1_scr[h0 + kh:h0 + kh + 8].reshape(8 * TB, 256),
                t2_ref[kh], preferred_element_type=f32)
        y2 = acc2 + b2_ref[...]
        yh2 = jnp.maximum(y2[:, :128], y2[:, 128:]).reshape(4, 2 * TB, 128)
        pooled_halves.append(
            jnp.maximum(jnp.maximum(yh2[:, :TB], yh2[:, TB:]),
                        0.0).astype(bf16))

    feats = jnp.concatenate(
        [ph[i] for ph in pooled_halves for i in range(4)], axis=1)
    h1 = jnp.maximum(
        jnp.dot(feats, w1_ref[...], preferred_element_type=f32) + fb1_ref[...],
        0.0)
    out = jnp.dot(h1.astype(bf16), w2_ref[...],
                  preferred_element_type=f32) + fb2_ref[...]
    o_ref[...] = out.astype(o_ref.dtype)


@jax.jit
def _forward(x_nchw, t1, b1, t2, b2, w1, fb1, w2, fb2):
    B, Cin, H, W = x_nchw.shape
    Bp = -(-B // TB) * TB

    t1p = jnp.pad(t1, ((0, 0), (0, 32), (0, 0)))
    t1p = t1p.at[0, 96, :].set(b1[0]).reshape(384, 512)
    ws = (t1p.astype(jnp.bfloat16), t2.astype(jnp.bfloat16), b2,
          w1.astype(jnp.bfloat16), fb1, w2.astype(jnp.bfloat16), fb2)

    xp = jnp.pad(x_nchw, ((0, Bp - B), (0, 0), (0, 0), (0, 0)))

    n_chunks = 1
    C = Bp // n_chunks

    def transform(xc):
        xh = jnp.transpose(xc, (2, 0, 3, 1)).reshape(H, C, W * Cin)
        xh = jnp.pad(xh, ((1, 1), (0, 0), (0, 0)))
        return jnp.concatenate(
            [xh, jnp.ones((H + 2, C, 1), xh.dtype),
             jnp.zeros((H + 2, C, 31), xh.dtype)], axis=2).astype(jnp.bfloat16)

    call = pl.pallas_call(
        _fwd_kernel,
        out_shape=jax.ShapeDtypeStruct((C, 128), jnp.float32),
        grid=(C // TB,),
        in_specs=[
            pl.BlockSpec((H + 2, TB, 128), lambda i: (0, i, 0)),
            pl.BlockSpec((384, 512), lambda i: (0, 0)),
            pl.BlockSpec((3, 256, 256), lambda i: (0, 0, 0)),
            pl.BlockSpec((1, 256), lambda i: (0, 0)),
            pl.BlockSpec((1024, 256), lambda i: (0, 0)),
            pl.BlockSpec((1, 256), lambda i: (0, 0)),
            pl.BlockSpec((256, 128), lambda i: (0, 0)),
            pl.BlockSpec((1, 128), lambda i: (0, 0)),
        ],
        out_specs=pl.BlockSpec((TB, 128), lambda i: (i, 0)),
        scratch_shapes=[pltpu.VMEM((18, TB, 256), jnp.bfloat16)],
        compiler_params=pltpu.CompilerParams(
            dimension_semantics=("parallel",),
            vmem_limit_bytes=VMEM_LIMIT_BYTES),
    )
    outs = [call(transform(xp[c * C:(c + 1) * C]), *ws)
            for c in range(n_chunks)]
    out = outs[0] if n_chunks == 1 else jnp.concatenate(outs, axis=0)
    return out[:B, :10]


def kernel(x, t1, b1, t2, b2, w1, fb1, w2, fb2):
    return _forward(x, t1, b1, t2, b2, w1, fb1, w2, fb2)


# speedup vs baseline: 1.4220x; 1.0004x over previous
import functools

import jax
import jax.numpy as jnp
from jax.experimental import pallas as pl
from jax.experimental.pallas import tpu as pltpu

TB = 512
VMEM_LIMIT_BYTES = 96 * 1024 * 1024


def _fwd_kernel(x_ref, t1_ref, t2_ref, b2_ref,
                w1_ref, fb1_ref, w2_ref, fb2_ref,
                o_ref, a1_scr):
    f32 = jnp.float32
    bf16 = jnp.bfloat16

    lhs = jnp.concatenate(
        [x_ref[kh:kh + 32].reshape(32 * TB, 128) for kh in range(3)], axis=1)
    zpad = jnp.zeros((1, TB, 256), bf16)
    a1_scr[0:1] = zpad
    a1_scr[17:18] = zpad
    for q in range(4):
        y = jnp.dot(lhs[q * 8 * TB:(q + 1) * 8 * TB], t1_ref[...],
                    preferred_element_type=f32)
        yh = jnp.maximum(y[:, :256], y[:, 256:]).reshape(4, 2 * TB, 256)
        a1 = jnp.maximum(jnp.maximum(yh[:, :TB], yh[:, TB:]), 0.0)
        a1_scr[1 + 4 * q:5 + 4 * q] = a1.astype(bf16)

    pooled_parts = []
    for q in range(4):
        h0 = 4 * q
        acc2 = jnp.dot(a1_scr[h0:h0 + 4].reshape(4 * TB, 256), t2_ref[0],
                       preferred_element_type=f32)
        for kh in (1, 2):
            acc2 = acc2 + jnp.dot(
                a1_scr[h0 + kh:h0 + kh + 4].reshape(4 * TB, 256),
                t2_ref[kh], preferred_element_type=f32)
        y2 = acc2 + b2_ref[...]
        yh2 = jnp.maximum(y2[:, :128], y2[:, 128:]).reshape(2, 2 * TB, 128)
        pooled_parts.append(
            jnp.maximum(jnp.maximum(yh2[:, :TB], yh2[:, TB:]),
                        0.0).astype(bf16))

    feats = jnp.concatenate(
        [ph[i] for ph in pooled_parts for i in range(2)], axis=1)
    h1 = jnp.maximum(
        jnp.dot(feats, w1_ref[...], preferred_element_type=f32) + fb1_ref[...],
        0.0)
    out = jnp.dot(h1.astype(bf16), w2_ref[...],
                  preferred_element_type=f32) + fb2_ref[...]
    o_ref[...] = out.astype(o_ref.dtype)


@jax.jit
def _forward(x_nchw, t1, b1, t2, b2, w1, fb1, w2, fb2):
    B, Cin, H, W = x_nchw.shape
    Bp = -(-B // TB) * TB

    t1p = jnp.pad(t1, ((0, 0), (0, 32), (0, 0)))
    t1p = t1p.at[0, 96, :].set(b1[0]).reshape(384, 512)
    ws = (t1p.astype(jnp.bfloat16), t2.astype(jnp.bfloat16), b2,
          w1.astype(jnp.bfloat16), fb1, w2.astype(jnp.bfloat16), fb2)

    xp = jnp.pad(x_nchw, ((0, Bp - B), (0, 0), (0, 0), (0, 0)))

    n_chunks = 1
    C = Bp // n_chunks

    def transform(xc):
        xh = jnp.transpose(xc, (2, 0, 3, 1)).reshape(H, C, W * Cin)
        xh = jnp.pad(xh, ((1, 1), (0, 0), (0, 0)))
        return jnp.concatenate(
            [xh, jnp.ones((H + 2, C, 1), xh.dtype),
             jnp.zeros((H + 2, C, 31), xh.dtype)], axis=2).astype(jnp.bfloat16)

    call = pl.pallas_call(
        _fwd_kernel,
        out_shape=jax.ShapeDtypeStruct((C, 128), jnp.float32),
        grid=(C // TB,),
        in_specs=[
            pl.BlockSpec((H + 2, TB, 128), lambda i: (0, i, 0)),
            pl.BlockSpec((384, 512), lambda i: (0, 0)),
            pl.BlockSpec((3, 256, 256), lambda i: (0, 0, 0)),
            pl.BlockSpec((1, 256), lambda i: (0, 0)),
            pl.BlockSpec((1024, 256), lambda i: (0, 0)),
            pl.BlockSpec((1, 256), lambda i: (0, 0)),
            pl.BlockSpec((256, 128), lambda i: (0, 0)),
            pl.BlockSpec((1, 128), lambda i: (0, 0)),
        ],
        out_specs=pl.BlockSpec((TB, 128), lambda i: (i, 0)),
        scratch_shapes=[pltpu.VMEM((18, TB, 256), jnp.bfloat16)],
        compiler_params=pltpu.CompilerParams(
            dimension_semantics=("parallel",),
            vmem_limit_bytes=VMEM_LIMIT_BYTES),
    )
    outs = [call(transform(xp[c * C:(c + 1) * C]), *ws)
            for c in range(n_chunks)]
    out = outs[0] if n_chunks == 1 else jnp.concatenate(outs, axis=0)
    return out[:B, :10]


def kernel(x, t1, b1, t2, b2, w1, fb1, w2, fb2):
    return _forward(x, t1, b1, t2, b2, w1, fb1, w2, fb2)


# speedup vs baseline: 1.5653x; 1.1008x over previous
import functools

import jax
import jax.numpy as jnp
from jax.experimental import pallas as pl
from jax.experimental.pallas import tpu as pltpu

TB = 512
VMEM_LIMIT_BYTES = 96 * 1024 * 1024


def _fwd_kernel(x_ref, t1_ref, t2_ref, b2_ref,
                w1_ref, fb1_ref, w2_ref, fb2_ref,
                o_ref, a1_scr):
    f32 = jnp.float32
    bf16 = jnp.bfloat16

    lhs = jnp.concatenate(
        [x_ref[kh:kh + 32].reshape(32 * TB, 128) for kh in range(3)], axis=1)
    zpad = jnp.zeros((1, TB, 256), bf16)
    a1_scr[0:1] = zpad
    a1_scr[17:18] = zpad
    for q in range(4):
        y = jnp.dot(lhs[q * 8 * TB:(q + 1) * 8 * TB], t1_ref[...],
                    preferred_element_type=f32)
        yh = jnp.maximum(y[:, :256], y[:, 256:]).reshape(4, 2 * TB, 256)
        a1 = jnp.maximum(jnp.maximum(yh[:, :TB], yh[:, TB:]), 0.0)
        a1_scr[1 + 4 * q:5 + 4 * q] = a1.astype(bf16)

    pooled_parts = []
    for q in range(4):
        h0 = 4 * q
        acc2 = jnp.dot(a1_scr[h0:h0 + 4].reshape(4 * TB, 256), t2_ref[0],
                       preferred_element_type=f32)
        for kh in (1, 2):
            acc2 = acc2 + jnp.dot(
                a1_scr[h0 + kh:h0 + kh + 4].reshape(4 * TB, 256),
                t2_ref[kh], preferred_element_type=f32)
        y2 = acc2 + b2_ref[...]
        yh2 = jnp.maximum(y2[:, :128], y2[:, 128:]).reshape(2, 2 * TB, 128)
        pooled_parts.append(
            jnp.maximum(jnp.maximum(yh2[:, :TB], yh2[:, TB:]),
                        0.0).astype(bf16))

    feats = jnp.concatenate(
        [ph[i] for ph in pooled_parts for i in range(2)], axis=1)
    h1 = jnp.maximum(
        jnp.dot(feats, w1_ref[...], preferred_element_type=f32) + fb1_ref[...],
        0.0)
    out = jnp.dot(h1.astype(bf16), w2_ref[...],
                  preferred_element_type=f32) + fb2_ref[...]
    o_ref[...] = out.astype(o_ref.dtype)


@jax.jit
def _forward(x_nchw, t1, b1, t2, b2, w1, fb1, w2, fb2):
    B, Cin, H, W = x_nchw.shape
    Bp = -(-B // TB) * TB

    r = jnp.arange(96)
    t1n = t1[:, (r % 32) * 3 + r // 32, :]
    t1p = jnp.pad(t1n, ((0, 0), (0, 32), (0, 0)))
    t1p = t1p.at[0, 96, :].set(b1[0]).reshape(384, 512)
    ws = (t1p.astype(jnp.bfloat16), t2.astype(jnp.bfloat16), b2,
          w1.astype(jnp.bfloat16), fb1, w2.astype(jnp.bfloat16), fb2)

    xp = jnp.pad(x_nchw, ((0, Bp - B), (0, 0), (0, 0), (0, 0)))

    n_chunks = 1
    C = Bp // n_chunks

    def transform(xc):
        xh = jnp.transpose(xc, (2, 0, 1, 3)).reshape(H, C, W * Cin)
        xh = jnp.pad(xh, ((1, 1), (0, 0), (0, 0)))
        return jnp.concatenate(
            [xh, jnp.ones((H + 2, C, 1), xh.dtype),
             jnp.zeros((H + 2, C, 31), xh.dtype)], axis=2).astype(jnp.bfloat16)

    call = pl.pallas_call(
        _fwd_kernel,
        out_shape=jax.ShapeDtypeStruct((C, 128), jnp.float32),
        grid=(C // TB,),
        in_specs=[
            pl.BlockSpec((H + 2, TB, 128), lambda i: (0, i, 0)),
            pl.BlockSpec((384, 512), lambda i: (0, 0)),
            pl.BlockSpec((3, 256, 256), lambda i: (0, 0, 0)),
            pl.BlockSpec((1, 256), lambda i: (0, 0)),
            pl.BlockSpec((1024, 256), lambda i: (0, 0)),
            pl.BlockSpec((1, 256), lambda i: (0, 0)),
            pl.BlockSpec((256, 128), lambda i: (0, 0)),
            pl.BlockSpec((1, 128), lambda i: (0, 0)),
        ],
        out_specs=pl.BlockSpec((TB, 128), lambda i: (i, 0)),
        scratch_shapes=[pltpu.VMEM((18, TB, 256), jnp.bfloat16)],
        compiler_params=pltpu.CompilerParams(
            dimension_semantics=("parallel",),
            vmem_limit_bytes=VMEM_LIMIT_BYTES),
    )
    outs = [call(transform(xp[c * C:(c + 1) * C]), *ws)
            for c in range(n_chunks)]
    out = outs[0] if n_chunks == 1 else jnp.concatenate(outs, axis=0)
    return out[:B, :10]


def kernel(x, t1, b1, t2, b2, w1, fb1, w2, fb2):
    return _forward(x, t1, b1, t2, b2, w1, fb1, w2, fb2)
